# baseline (speedup 1.0000x reference)
import functools

import jax
import jax.numpy as jnp
from jax import lax
from jax.experimental import pallas as pl
from jax.experimental.pallas import tpu as pltpu
from jax.experimental.pallas import tpu_sc as plsc

N = 10000
E = 320000
D = 128
H = 128

NC = 2
NS = 16
EW = E // (NC * NS)
K = 40
NCHUNK = EW // K
NP = 10240
ROWS_PER_TILE = NP // NS
ZR = 32
NBUF = 5
NW = NC * NS


def _sc_body(x_hbm, ei_hbm, agg_out, cnt_out,
             zrow, czero, ones1, rows, agg_s, cnt_s, *rest):
    idxb = rest[0:NBUF]
    isem = rest[NBUF:2 * NBUF]
    gsem = rest[2 * NBUF:3 * NBUF]
    ssem = rest[3 * NBUF:4 * NBUF]
    zsem = rest[4 * NBUF]
    c = lax.axis_index("c")
    s = lax.axis_index("s")
    wid = c * NS + s

    zero16 = jnp.zeros((16,), jnp.float32)
    one16 = jnp.ones((16,), jnp.float32)

    def fill_zrow(i, _):
        for j in range(D // 16):
            zrow[i, pl.ds(j * 16, 16)] = zero16
        return 0
    lax.fori_loop(0, ZR, fill_zrow, 0)

    def fill_czero(i, _):
        czero[pl.ds(i * 16, 16)] = zero16
        return 0
    lax.fori_loop(0, ROWS_PER_TILE // 16, fill_czero, 0)

    for off in sorted(set(list(range(0, K - 15, 16)) + [K - 16])):
        ones1[pl.ds(off, 16)] = one16

    r0 = s * ROWS_PER_TILE
    base = wid * EW
    d_cz = pltpu.async_copy(czero, cnt_s.at[pl.ds(r0, ROWS_PER_TILE)], zsem)

    def zero_agg(i, _):
        pltpu.async_copy(zrow, agg_s.at[pl.ds(r0 + i * ZR, ZR)], zsem)
        return 0
    lax.fori_loop(0, ROWS_PER_TILE // ZR, zero_agg, 0)

    def zero_agg_wait(i, _):
        pltpu.make_async_copy(zrow, agg_s.at[pl.ds(r0, ZR)], zsem).wait()
        return 0
    lax.fori_loop(0, ROWS_PER_TILE // ZR, zero_agg_wait, 0)
    d_cz.wait()

    plsc.subcore_barrier()

    base2 = wid * NCHUNK * 2 * K

    def src_of(b):
        return idxb[b].at[pl.ds(0, K)]

    def dst_of(b):
        return idxb[b].at[pl.ds(K, K)]

    def load_idx(j, b):
        pltpu.async_copy(ei_hbm.at[pl.ds(base2 + j * 2 * K, 2 * K)], idxb[b],
                         isem[b])

    def wait_idx(b):
        pltpu.make_async_copy(ei_hbm.at[pl.ds(base2, 2 * K)], idxb[b],
                              isem[b]).wait()

    def issue_gather(b):
        pltpu.async_copy(x_hbm.at[src_of(b)], rows.at[b], gsem[b])

    def wait_gather(b):
        pltpu.make_async_copy(x_hbm.at[src_of(b)], rows.at[b],
                              gsem[b]).wait()

    def issue_scatter(b):
        pltpu.async_copy(rows.at[b], agg_s.at[dst_of(b)], ssem[b], add=True)
        pltpu.async_copy(ones1, cnt_s.at[dst_of(b)], ssem[b], add=True)

    def wait_scatter(b):
        pltpu.make_async_copy(rows.at[b], agg_s.at[dst_of(b)],
                              ssem[b]).wait()
        pltpu.make_async_copy(ones1, cnt_s.at[dst_of(b)], ssem[b]).wait()

    def emit_slot(i, b, load_ok, gather_ok, drain_ok):
        if load_ok:
            load_idx(i + 4, (b + 4) % NBUF)
        if gather_ok:
            b2 = (b + 2) % NBUF
            wait_idx(b2)
            issue_gather(b2)
        wait_gather(b)
        issue_scatter(b)
        if drain_ok:
            wait_scatter((b - 1) % NBUF)

    for j in range(4):
        load_idx(j, j)
    for j in range(2):
        wait_idx(j)
        issue_gather(j)

    for b in range(NBUF):
        emit_slot(b, b, True, True, b >= 1)

    def outer(g, _):
        for b in range(NBUF):
            emit_slot(g * NBUF + b, b, True, True, True)
        return 0
    lax.fori_loop(1, NCHUNK // NBUF - 1, outer, 0)

    for b in range(NBUF):
        i = NCHUNK - NBUF + b
        emit_slot(i, b, i + 4 < NCHUNK, i + 2 < NCHUNK, True)

    wait_scatter((NCHUNK - 1) % NBUF)

    plsc.subcore_barrier()

    pltpu.sync_copy(agg_s.at[pl.ds(r0, ROWS_PER_TILE)],
                    agg_out.at[c, pl.ds(r0, ROWS_PER_TILE)])
    pltpu.sync_copy(cnt_s.at[pl.ds(r0, ROWS_PER_TILE)],
                    cnt_out.at[pl.ds(c * NP + r0, ROWS_PER_TILE)])


@functools.lru_cache(maxsize=1)
def _sc_agg():
    return pl.kernel(
        _sc_body,
        mesh=plsc.VectorSubcoreMesh(core_axis_name="c", subcore_axis_name="s",
                                    num_cores=NC, num_subcores=NS),
        out_type=[jax.ShapeDtypeStruct((NC, NP, D), jnp.float32),
                  jax.ShapeDtypeStruct((NC * NP,), jnp.float32)],
        scratch_types=[
            pltpu.VMEM((ZR, D), jnp.float32),
            pltpu.VMEM((ROWS_PER_TILE,), jnp.float32),
            pltpu.VMEM((K,), jnp.float32),
            pltpu.VMEM((NBUF, K, D), jnp.float32),
            pltpu.VMEM_SHARED((NP, D), jnp.float32),
            pltpu.VMEM_SHARED((NP,), jnp.float32),
        ] + [pltpu.VMEM((2 * K,), jnp.int32)] * NBUF
          + [pltpu.SemaphoreType.DMA] * (3 * NBUF + 1),
    )


def _tc_body(agg_ref, cnt_ref, x_ref, wl_ref, bl_ref, wr_ref, wp_ref, bp_ref,
             ws_ref, bs_ref, outp_ref, outs_ref):
    p = agg_ref[0] + agg_ref[1]
    cnt = cnt_ref[0] + cnt_ref[1]
    inv = 1.0 / jnp.maximum(cnt, 1.0)
    aggm = p * inv
    h = jnp.dot(aggm, wl_ref[...], preferred_element_type=jnp.float32)
    h = h + bl_ref[...]
    h = h + jnp.dot(x_ref[...], wr_ref[...], preferred_element_type=jnp.float32)
    h = jnp.maximum(h, 0.0)

    def lsm(z):
        m = jnp.max(z, axis=1, keepdims=True)
        zz = z - m
        return zz - jnp.log(jnp.sum(jnp.exp(zz), axis=1, keepdims=True))

    lp = jnp.dot(h, wp_ref[...], preferred_element_type=jnp.float32)
    outp_ref[...] = lsm(lp + bp_ref[...])
    ls = jnp.dot(h, ws_ref[...], preferred_element_type=jnp.float32)
    outs_ref[...] = lsm(ls + bs_ref[...])


def _tc_head(agg_p, cnt_p, x, W_l, b_l, W_r, W_p, b_p, W_s, b_s):
    B = 5000
    grid = (N // B,)
    return pl.pallas_call(
        _tc_body,
        grid=grid,
        in_specs=[
            pl.BlockSpec((NC, B, D), lambda i: (0, i, 0)),
            pl.BlockSpec((NC, B, 1), lambda i: (0, i, 0)),
            pl.BlockSpec((B, D), lambda i: (i, 0)),
            pl.BlockSpec((D, H), lambda i: (0, 0)),
            pl.BlockSpec((1, H), lambda i: (0, 0)),
            pl.BlockSpec((D, H), lambda i: (0, 0)),
            pl.BlockSpec((H, 7), lambda i: (0, 0)),
            pl.BlockSpec((1, 7), lambda i: (0, 0)),
            pl.BlockSpec((H, 6), lambda i: (0, 0)),
            pl.BlockSpec((1, 6), lambda i: (0, 0)),
        ],
        out_specs=[pl.BlockSpec((B, 7), lambda i: (i, 0)),
                   pl.BlockSpec((B, 6), lambda i: (i, 0))],
        out_shape=[jax.ShapeDtypeStruct((N, 7), jnp.float32),
                   jax.ShapeDtypeStruct((N, 6), jnp.float32)],
    )(agg_p, cnt_p, x, W_l, b_l, W_r, W_p, b_p, W_s, b_s)


def kernel(x, edge_index, W_l, b_l, W_r, W_p, b_p, W_s, b_s):
    ei_packed = edge_index.reshape(2, NW, NCHUNK, K).transpose(
        1, 2, 0, 3).reshape(2 * E)
    agg_p, cnt_p = _sc_agg()(x, ei_packed)
    cnt_p = cnt_p.reshape(NC, NP, 1)
    outp, outs = _tc_head(agg_p, cnt_p, x, W_l, b_l.reshape(1, H), W_r,
                          W_p, b_p.reshape(1, 7), W_s, b_s.reshape(1, 6))
    return (outp, outs)


# speedup vs baseline: 1.2840x; 1.2840x over previous
import functools

import jax
import jax.numpy as jnp
from jax import lax
from jax.experimental import pallas as pl
from jax.experimental.pallas import tpu as pltpu
from jax.experimental.pallas import tpu_sc as plsc

N = 10000
E = 320000
D = 128
H = 128

NC = 2
NS = 16
EW = E // (NC * NS)
K = 40
NCHUNK = EW // K
NP = 10240
ROWS_PER_TILE = NP // NS
ZR = 32
NBUF = 5
NW = NC * NS


def _sc_body(x_hbm, ei_hbm, agg_out, cnt_out,
             zrow, czero, ones1, rows, agg_s, cnt_s, *rest):
    idxb = rest[0:NBUF]
    isem = rest[NBUF:2 * NBUF]
    gsem = rest[2 * NBUF:3 * NBUF]
    ssem = rest[3 * NBUF:4 * NBUF]
    zsem = rest[4 * NBUF]
    c = lax.axis_index("c")
    s = lax.axis_index("s")
    wid = c * NS + s

    zero16 = jnp.zeros((16,), jnp.float32)
    one16 = jnp.ones((16,), jnp.float32)

    def fill_zrow(i, _):
        for j in range(D // 16):
            zrow[i, pl.ds(j * 16, 16)] = zero16
        return 0
    lax.fori_loop(0, ZR, fill_zrow, 0)

    def fill_czero(i, _):
        czero[pl.ds(i * 16, 16)] = zero16
        return 0
    lax.fori_loop(0, ROWS_PER_TILE // 16, fill_czero, 0)

    for off in sorted(set(list(range(0, K - 15, 16)) + [K - 16])):
        ones1[pl.ds(off, 16)] = one16

    r0 = s * ROWS_PER_TILE
    base = wid * EW
    d_cz = pltpu.async_copy(czero, cnt_s.at[pl.ds(r0, ROWS_PER_TILE)], zsem)

    def zero_agg(i, _):
        pltpu.async_copy(zrow, agg_s.at[pl.ds(r0 + i * ZR, ZR)], zsem)
        return 0
    lax.fori_loop(0, ROWS_PER_TILE // ZR, zero_agg, 0)

    def zero_agg_wait(i, _):
        pltpu.make_async_copy(zrow, agg_s.at[pl.ds(r0, ZR)], zsem).wait()
        return 0
    lax.fori_loop(0, ROWS_PER_TILE // ZR, zero_agg_wait, 0)
    d_cz.wait()

    plsc.subcore_barrier()

    def src_of(b):
        return idxb[b].at[pl.ds(0, K)]

    def dst_of(b):
        return idxb[b].at[pl.ds(K, K)]

    def load_idx(j, b):
        pltpu.async_copy(ei_hbm.at[pl.ds(base + j * K, K)], src_of(b),
                         isem[b])
        pltpu.async_copy(ei_hbm.at[pl.ds(E + base + j * K, K)], dst_of(b),
                         isem[b])

    def wait_idx(b):
        pltpu.make_async_copy(ei_hbm.at[pl.ds(base, K)], src_of(b),
                              isem[b]).wait()
        pltpu.make_async_copy(ei_hbm.at[pl.ds(base, K)], dst_of(b),
                              isem[b]).wait()

    def issue_gather(b):
        pltpu.async_copy(x_hbm.at[src_of(b)], rows.at[b], gsem[b])

    def wait_gather(b):
        pltpu.make_async_copy(x_hbm.at[src_of(b)], rows.at[b],
                              gsem[b]).wait()

    def issue_scatter(b):
        pltpu.async_copy(rows.at[b], agg_s.at[dst_of(b)], ssem[b], add=True)
        pltpu.async_copy(ones1, cnt_s.at[dst_of(b)], ssem[b], add=True)

    def wait_scatter(b):
        pltpu.make_async_copy(rows.at[b], agg_s.at[dst_of(b)],
                              ssem[b]).wait()
        pltpu.make_async_copy(ones1, cnt_s.at[dst_of(b)], ssem[b]).wait()

    def emit_slot(i, b, load_ok, gather_ok, drain_ok):
        if load_ok:
            load_idx(i + 4, (b + 4) % NBUF)
        if gather_ok:
            b2 = (b + 2) % NBUF
            wait_idx(b2)
            issue_gather(b2)
        wait_gather(b)
        issue_scatter(b)
        if drain_ok:
            wait_scatter((b - 1) % NBUF)

    for j in range(4):
        load_idx(j, j)
    for j in range(2):
        wait_idx(j)
        issue_gather(j)

    for b in range(NBUF):
        emit_slot(b, b, True, True, b >= 1)

    def outer(g, _):
        for b in range(NBUF):
            emit_slot(g * NBUF + b, b, True, True, True)
        return 0
    lax.fori_loop(1, NCHUNK // NBUF - 1, outer, 0)

    for b in range(NBUF):
        i = NCHUNK - NBUF + b
        emit_slot(i, b, i + 4 < NCHUNK, i + 2 < NCHUNK, True)

    wait_scatter((NCHUNK - 1) % NBUF)

    plsc.subcore_barrier()

    pltpu.sync_copy(agg_s.at[pl.ds(r0, ROWS_PER_TILE)],
                    agg_out.at[c, pl.ds(r0, ROWS_PER_TILE)])
    pltpu.sync_copy(cnt_s.at[pl.ds(r0, ROWS_PER_TILE)],
                    cnt_out.at[pl.ds(c * NP + r0, ROWS_PER_TILE)])


@functools.lru_cache(maxsize=1)
def _sc_agg():
    return pl.kernel(
        _sc_body,
        mesh=plsc.VectorSubcoreMesh(core_axis_name="c", subcore_axis_name="s",
                                    num_cores=NC, num_subcores=NS),
        out_type=[jax.ShapeDtypeStruct((NC, NP, D), jnp.float32),
                  jax.ShapeDtypeStruct((NC * NP,), jnp.float32)],
        scratch_types=[
            pltpu.VMEM((ZR, D), jnp.float32),
            pltpu.VMEM((ROWS_PER_TILE,), jnp.float32),
            pltpu.VMEM((K,), jnp.float32),
            pltpu.VMEM((NBUF, K, D), jnp.float32),
            pltpu.VMEM_SHARED((NP, D), jnp.float32),
            pltpu.VMEM_SHARED((NP,), jnp.float32),
        ] + [pltpu.VMEM((2 * K,), jnp.int32)] * NBUF
          + [pltpu.SemaphoreType.DMA] * (3 * NBUF + 1),
    )


def _tc_body(agg_ref, cnt_ref, x_ref, wl_ref, bl_ref, wr_ref, wp_ref, bp_ref,
             ws_ref, bs_ref, outp_ref, outs_ref):
    p = agg_ref[0] + agg_ref[1]
    cnt = cnt_ref[0] + cnt_ref[1]
    inv = 1.0 / jnp.maximum(cnt, 1.0)
    aggm = p * inv
    h = jnp.dot(aggm, wl_ref[...], preferred_element_type=jnp.float32)
    h = h + bl_ref[...]
    h = h + jnp.dot(x_ref[...], wr_ref[...], preferred_element_type=jnp.float32)
    h = jnp.maximum(h, 0.0)

    def lsm(z):
        m = jnp.max(z, axis=1, keepdims=True)
        zz = z - m
        return zz - jnp.log(jnp.sum(jnp.exp(zz), axis=1, keepdims=True))

    lp = jnp.dot(h, wp_ref[...], preferred_element_type=jnp.float32)
    outp_ref[...] = lsm(lp + bp_ref[...])
    ls = jnp.dot(h, ws_ref[...], preferred_element_type=jnp.float32)
    outs_ref[...] = lsm(ls + bs_ref[...])


def _tc_head(agg_p, cnt_p, x, W_l, b_l, W_r, W_p, b_p, W_s, b_s):
    B = 2000
    grid = (N // B,)
    return pl.pallas_call(
        _tc_body,
        grid=grid,
        in_specs=[
            pl.BlockSpec((NC, B, D), lambda i: (0, i, 0)),
            pl.BlockSpec((NC, B, 1), lambda i: (0, i, 0)),
            pl.BlockSpec((B, D), lambda i: (i, 0)),
            pl.BlockSpec((D, H), lambda i: (0, 0)),
            pl.BlockSpec((1, H), lambda i: (0, 0)),
            pl.BlockSpec((D, H), lambda i: (0, 0)),
            pl.BlockSpec((H, 7), lambda i: (0, 0)),
            pl.BlockSpec((1, 7), lambda i: (0, 0)),
            pl.BlockSpec((H, 6), lambda i: (0, 0)),
            pl.BlockSpec((1, 6), lambda i: (0, 0)),
        ],
        out_specs=[pl.BlockSpec((B, 7), lambda i: (i, 0)),
                   pl.BlockSpec((B, 6), lambda i: (i, 0))],
        out_shape=[jax.ShapeDtypeStruct((N, 7), jnp.float32),
                   jax.ShapeDtypeStruct((N, 6), jnp.float32)],
    )(agg_p, cnt_p, x, W_l, b_l, W_r, W_p, b_p, W_s, b_s)


def kernel(x, edge_index, W_l, b_l, W_r, W_p, b_p, W_s, b_s):
    agg_p, cnt_p = _sc_agg()(x, edge_index.reshape(2 * E))
    cnt_p = cnt_p.reshape(NC, NP, 1)
    outp, outs = _tc_head(agg_p, cnt_p, x, W_l, b_l.reshape(1, H), W_r,
                          W_p, b_p.reshape(1, 7), W_s, b_s.reshape(1, 6))
    return (outp, outs)


# speedup vs baseline: 1.3451x; 1.0476x over previous
import functools

import jax
import jax.numpy as jnp
from jax import lax
from jax.experimental import pallas as pl
from jax.experimental.pallas import tpu as pltpu
from jax.experimental.pallas import tpu_sc as plsc

N = 10000
E = 320000
D = 128
H = 128

NC = 2
NS = 16
EW = E // (NC * NS)
K = 40
NCHUNK = EW // K
NP = 10240
ROWS_PER_TILE = NP // NS
ZR = 32
NBUF = 5
NW = NC * NS


def _sc_body(x_hbm, ei_hbm, agg_out, cnt_out,
             zrow, czero, ones1, rows, agg_s, cnt_s, *rest):
    idxb = rest[0:2 * NBUF]
    isem = rest[2 * NBUF:4 * NBUF]
    gsem = rest[4 * NBUF:5 * NBUF]
    ssem = rest[5 * NBUF:6 * NBUF]
    zsem = rest[6 * NBUF]
    c = lax.axis_index("c")
    s = lax.axis_index("s")
    wid = c * NS + s

    zero16 = jnp.zeros((16,), jnp.float32)
    one16 = jnp.ones((16,), jnp.float32)

    def fill_zrow(i, _):
        for j in range(D // 16):
            zrow[i, pl.ds(j * 16, 16)] = zero16
        return 0
    lax.fori_loop(0, ZR, fill_zrow, 0)

    def fill_czero(i, _):
        czero[pl.ds(i * 16, 16)] = zero16
        return 0
    lax.fori_loop(0, ROWS_PER_TILE // 16, fill_czero, 0)

    for off in sorted(set(list(range(0, K - 15, 16)) + [K - 16])):
        ones1[pl.ds(off, 16)] = one16

    r0 = s * ROWS_PER_TILE
    base = wid * EW
    d_cz = pltpu.async_copy(czero, cnt_s.at[pl.ds(r0, ROWS_PER_TILE)], zsem)

    def zero_agg(i, _):
        pltpu.async_copy(zrow, agg_s.at[pl.ds(r0 + i * ZR, ZR)], zsem)
        return 0
    lax.fori_loop(0, ROWS_PER_TILE // ZR, zero_agg, 0)

    def zero_agg_wait(i, _):
        pltpu.make_async_copy(zrow, agg_s.at[pl.ds(r0, ZR)], zsem).wait()
        return 0
    lax.fori_loop(0, ROWS_PER_TILE // ZR, zero_agg_wait, 0)
    d_cz.wait()

    plsc.subcore_barrier()

    def src_of(b):
        return idxb[b].at[pl.ds(0, K)]

    def dst_of(b):
        return idxb[b].at[pl.ds(K, K)]

    def load_idx(j, b):
        pltpu.async_copy(ei_hbm.at[pl.ds(base + j * K, K)], src_of(b),
                         isem[b])
        pltpu.async_copy(ei_hbm.at[pl.ds(E + base + j * K, K)], dst_of(b),
                         isem[b])

    def wait_idx(b):
        pltpu.make_async_copy(ei_hbm.at[pl.ds(base, K)], src_of(b),
                              isem[b]).wait()
        pltpu.make_async_copy(ei_hbm.at[pl.ds(base, K)], dst_of(b),
                              isem[b]).wait()

    def issue_gather(bi, br):
        pltpu.async_copy(x_hbm.at[src_of(bi)], rows.at[br], gsem[br])

    def wait_gather(br):
        pltpu.make_async_copy(x_hbm.at[src_of(0)], rows.at[br],
                              gsem[br]).wait()

    def issue_scatter(br, bi):
        pltpu.async_copy(rows.at[br], agg_s.at[dst_of(bi)], ssem[br],
                         add=True)
        pltpu.async_copy(ones1, cnt_s.at[dst_of(bi)], ssem[br], add=True)

    def wait_scatter(br, bi):
        pltpu.make_async_copy(rows.at[br], agg_s.at[dst_of(bi)],
                              ssem[br]).wait()
        pltpu.make_async_copy(ones1, cnt_s.at[dst_of(bi)], ssem[br]).wait()

    for j in range(6):
        load_idx(j, j)
    for j in range(3):
        wait_idx(j)
        issue_gather(j, j % NBUF)

    def outer(g, _):
        for b in range(2 * NBUF):
            i = g * 2 * NBUF + b
            b5 = b % NBUF
            bl = (b + 6) % (2 * NBUF)
            bg = (b + 3) % (2 * NBUF)

            @pl.when(i + 6 < NCHUNK)
            def _load():
                load_idx(i + 6, bl)

            @pl.when(i >= 2)
            def _drain():
                wait_scatter((b - 2) % NBUF, (b - 2) % (2 * NBUF))

            @pl.when(i + 3 < NCHUNK)
            def _gather():
                wait_idx(bg)
                issue_gather(bg, (b + 3) % NBUF)

            wait_gather(b5)
            issue_scatter(b5, b)
        return 0
    lax.fori_loop(0, NCHUNK // (2 * NBUF), outer, 0)

    wait_scatter((NCHUNK - 2) % NBUF, (NCHUNK - 2) % (2 * NBUF))
    wait_scatter((NCHUNK - 1) % NBUF, (NCHUNK - 1) % (2 * NBUF))

    plsc.subcore_barrier()

    pltpu.sync_copy(agg_s.at[pl.ds(r0, ROWS_PER_TILE)],
                    agg_out.at[c, pl.ds(r0, ROWS_PER_TILE)])
    pltpu.sync_copy(cnt_s.at[pl.ds(r0, ROWS_PER_TILE)],
                    cnt_out.at[pl.ds(c * NP + r0, ROWS_PER_TILE)])


@functools.lru_cache(maxsize=1)
def _sc_agg():
    return pl.kernel(
        _sc_body,
        mesh=plsc.VectorSubcoreMesh(core_axis_name="c", subcore_axis_name="s",
                                    num_cores=NC, num_subcores=NS),
        out_type=[jax.ShapeDtypeStruct((NC, NP, D), jnp.float32),
                  jax.ShapeDtypeStruct((NC * NP,), jnp.float32)],
        scratch_types=[
            pltpu.VMEM((ZR, D), jnp.float32),
            pltpu.VMEM((ROWS_PER_TILE,), jnp.float32),
            pltpu.VMEM((K,), jnp.float32),
            pltpu.VMEM((NBUF, K, D), jnp.float32),
            pltpu.VMEM_SHARED((NP, D), jnp.float32),
            pltpu.VMEM_SHARED((NP,), jnp.float32),
        ] + [pltpu.VMEM((2 * K,), jnp.int32)] * (2 * NBUF)
          + [pltpu.SemaphoreType.DMA] * (4 * NBUF + 1),
    )


def _tc_body(agg_ref, cnt_ref, x_ref, wl_ref, bl_ref, wr_ref, wp_ref, bp_ref,
             ws_ref, bs_ref, outp_ref, outs_ref):
    p = agg_ref[0] + agg_ref[1]
    cnt = cnt_ref[0] + cnt_ref[1]
    inv = 1.0 / jnp.maximum(cnt, 1.0)
    aggm = p * inv
    h = jnp.dot(aggm, wl_ref[...], preferred_element_type=jnp.float32)
    h = h + bl_ref[...]
    h = h + jnp.dot(x_ref[...], wr_ref[...], preferred_element_type=jnp.float32)
    h = jnp.maximum(h, 0.0)

    def lsm(z):
        m = jnp.max(z, axis=1, keepdims=True)
        zz = z - m
        return zz - jnp.log(jnp.sum(jnp.exp(zz), axis=1, keepdims=True))

    lp = jnp.dot(h, wp_ref[...], preferred_element_type=jnp.float32)
    outp_ref[...] = lsm(lp + bp_ref[...])
    ls = jnp.dot(h, ws_ref[...], preferred_element_type=jnp.float32)
    outs_ref[...] = lsm(ls + bs_ref[...])


def _tc_head(agg_p, cnt_p, x, W_l, b_l, W_r, W_p, b_p, W_s, b_s):
    B = 2000
    grid = (N // B,)
    return pl.pallas_call(
        _tc_body,
        grid=grid,
        in_specs=[
            pl.BlockSpec((NC, B, D), lambda i: (0, i, 0)),
            pl.BlockSpec((NC, B, 1), lambda i: (0, i, 0)),
            pl.BlockSpec((B, D), lambda i: (i, 0)),
            pl.BlockSpec((D, H), lambda i: (0, 0)),
            pl.BlockSpec((1, H), lambda i: (0, 0)),
            pl.BlockSpec((D, H), lambda i: (0, 0)),
            pl.BlockSpec((H, 7), lambda i: (0, 0)),
            pl.BlockSpec((1, 7), lambda i: (0, 0)),
            pl.BlockSpec((H, 6), lambda i: (0, 0)),
            pl.BlockSpec((1, 6), lambda i: (0, 0)),
        ],
        out_specs=[pl.BlockSpec((B, 7), lambda i: (i, 0)),
                   pl.BlockSpec((B, 6), lambda i: (i, 0))],
        out_shape=[jax.ShapeDtypeStruct((N, 7), jnp.float32),
                   jax.ShapeDtypeStruct((N, 6), jnp.float32)],
    )(agg_p, cnt_p, x, W_l, b_l, W_r, W_p, b_p, W_s, b_s)


def kernel(x, edge_index, W_l, b_l, W_r, W_p, b_p, W_s, b_s):
    agg_p, cnt_p = _sc_agg()(x, edge_index.reshape(2 * E))
    cnt_p = cnt_p.reshape(NC, NP, 1)
    outp, outs = _tc_head(agg_p, cnt_p, x, W_l, b_l.reshape(1, H), W_r,
                          W_p, b_p.reshape(1, 7), W_s, b_s.reshape(1, 6))
    return (outp, outs)
